# Initial kernel scaffold; baseline (speedup 1.0000x reference)
import jax
import jax.numpy as jnp
from jax.experimental import pallas as pl


def kernel(x_nchw, w_deconv, gamma, beta):
    raise NotImplementedError("write your pallas kernel here")


# speedup vs baseline: 3.8159x; 3.8159x over previous
import functools

import jax
import jax.numpy as jnp
from jax.experimental import pallas as pl
from jax.experimental.pallas import tpu as pltpu

_SLOPE = 0.02
_EPS = 1e-5


def _conv_stats_kernel(x_ref, t_ref, y_ref, s1_ref, s2_ref, *, H):
    NB = x_ref.shape[0]
    WC = x_ref.shape[2]
    R = NB * H

    x = x_ref[...].reshape(R, WC)
    a = jnp.where(x >= 0, x, x * _SLOPE)

    row = jax.lax.broadcasted_iota(jnp.int32, (R, 1), 0)
    zrow = jnp.zeros((1, WC), dtype=a.dtype)
    a_dn = jnp.concatenate([zrow, a[:R - 1]], axis=0)
    a_dn = jnp.where(row % H == 0, jnp.zeros_like(a_dn), a_dn)
    a_up = jnp.concatenate([a[1:], zrow], axis=0)
    a_up = jnp.where(row % H == H - 1, jnp.zeros_like(a_up), a_up)

    af = jnp.concatenate([a_dn, a, a_up], axis=1)
    y = jnp.dot(af, t_ref[...], preferred_element_type=jnp.float32)

    y_ref[...] = y.astype(y_ref.dtype).reshape(NB, H, WC)
    s1_ref[...] = jnp.sum(y, axis=0).reshape(1, 1, WC)
    s2_ref[...] = jnp.sum(y * y, axis=0).reshape(1, 1, WC)


def _bn_apply_kernel(y_ref, scale_ref, shift_ref, out_ref):
    out_ref[...] = y_ref[...].astype(jnp.float32) * scale_ref[...] + shift_ref[...]


def _stacked_toeplitz(w_deconv, W):
    wc = jnp.transpose(jnp.flip(w_deconv, axis=(2, 3)), (2, 3, 0, 1))
    wc = wc.astype(jnp.float32)
    bands = jnp.stack([jnp.eye(W, k=1 - kw, dtype=jnp.float32)
                       for kw in range(3)])
    t = jnp.einsum('kab,rkcd->racbd', bands, wc)
    C = w_deconv.shape[0]
    return t.reshape(3 * W * C, W * C).astype(jnp.bfloat16)


@jax.jit
def _forward(x_nchw, w_deconv, gamma, beta):
    N, C, H, W = x_nchw.shape
    WC = W * C
    NB = 32
    while N % NB:
        NB //= 2
    G = N // NB

    x2 = jnp.transpose(x_nchw, (0, 2, 3, 1)).reshape(N, H, WC)
    x2 = x2.astype(jnp.bfloat16)
    t = _stacked_toeplitz(w_deconv, W)

    cparams = pltpu.CompilerParams(
        dimension_semantics=("parallel",),
        vmem_limit_bytes=64 * 1024 * 1024,
    )

    conv_kernel = functools.partial(_conv_stats_kernel, H=H)
    y, s1, s2 = pl.pallas_call(
        conv_kernel,
        grid=(G,),
        in_specs=[
            pl.BlockSpec((NB, H, WC), lambda g: (g, 0, 0)),
            pl.BlockSpec((3 * WC, WC), lambda g: (0, 0)),
        ],
        out_specs=(
            pl.BlockSpec((NB, H, WC), lambda g: (g, 0, 0)),
            pl.BlockSpec((1, 1, WC), lambda g: (g, 0, 0)),
            pl.BlockSpec((1, 1, WC), lambda g: (g, 0, 0)),
        ),
        out_shape=(
            jax.ShapeDtypeStruct((N, H, WC), jnp.bfloat16),
            jax.ShapeDtypeStruct((G, 1, WC), jnp.float32),
            jax.ShapeDtypeStruct((G, 1, WC), jnp.float32),
        ),
        compiler_params=cparams,
    )(x2, t)

    m_total = float(N * H * W)
    sum_c = jnp.sum(s1.reshape(G, W, C), axis=(0, 1))
    sq_c = jnp.sum(s2.reshape(G, W, C), axis=(0, 1))
    mean = sum_c / m_total
    var = jnp.maximum(sq_c / m_total - mean * mean, 0.0)
    inv = jax.lax.rsqrt(var + _EPS)
    scale_c = gamma.astype(jnp.float32) * inv
    shift_c = beta.astype(jnp.float32) - mean * scale_c
    scale_wc = jnp.tile(scale_c, W).reshape(1, WC)
    shift_wc = jnp.tile(shift_c, W).reshape(1, WC)

    out2 = pl.pallas_call(
        _bn_apply_kernel,
        grid=(G,),
        in_specs=[
            pl.BlockSpec((NB, H, WC), lambda g: (g, 0, 0)),
            pl.BlockSpec((1, WC), lambda g: (0, 0)),
            pl.BlockSpec((1, WC), lambda g: (0, 0)),
        ],
        out_specs=pl.BlockSpec((NB, H, WC), lambda g: (g, 0, 0)),
        out_shape=jax.ShapeDtypeStruct((N, H, WC), jnp.float32),
        compiler_params=cparams,
    )(y, scale_wc, shift_wc)

    return jnp.transpose(out2.reshape(N, H, W, C), (0, 3, 1, 2))


def kernel(x_nchw, w_deconv, gamma, beta):
    return _forward(x_nchw, w_deconv, gamma, beta)


# speedup vs baseline: 4.3757x; 1.1467x over previous
import functools

import numpy as np

import jax
import jax.numpy as jnp
from jax.experimental import pallas as pl
from jax.experimental.pallas import tpu as pltpu

_SLOPE = 0.02
_EPS = 1e-5


def _conv_stats_kernel(x_ref, t_ref, y_ref, s1_ref, s2_ref, *, H, C, QW, PW):
    NB = x_ref.shape[0]
    WC = x_ref.shape[2]
    R = NB * H
    Q = WC // (QW * C)
    PAD = ((PW - QW) // 2) * C

    x = x_ref[...].reshape(R, WC)
    a = jnp.where(x >= 0, x, x * _SLOPE)

    row = jax.lax.broadcasted_iota(jnp.int32, (R, 1), 0)
    zrow = jnp.zeros((1, WC), dtype=a.dtype)
    a_dn = jnp.concatenate([zrow, a[:R - 1]], axis=0)
    a_dn = jnp.where(row % H == 0, jnp.zeros_like(a_dn), a_dn)
    a_up = jnp.concatenate([a[1:], zrow], axis=0)
    a_up = jnp.where(row % H == H - 1, jnp.zeros_like(a_up), a_up)

    if PAD:
        zpad = jnp.zeros((R, PAD), dtype=a.dtype)
        ap_dn = jnp.concatenate([zpad, a_dn, zpad], axis=1)
        ap_md = jnp.concatenate([zpad, a, zpad], axis=1)
        ap_up = jnp.concatenate([zpad, a_up, zpad], axis=1)
    else:
        ap_dn, ap_md, ap_up = a_dn, a, a_up

    seg = PW * C
    ys = []
    for q in range(Q):
        lo = q * QW * C
        lhs = jnp.concatenate([ap_dn[:, lo:lo + seg],
                               ap_md[:, lo:lo + seg],
                               ap_up[:, lo:lo + seg]], axis=1)
        ys.append(jnp.dot(lhs, t_ref[q], preferred_element_type=jnp.float32))
    y = jnp.concatenate(ys, axis=1)

    y_ref[...] = y.astype(y_ref.dtype).reshape(NB, H, WC)
    s1_ref[...] = jnp.sum(y, axis=0).reshape(1, 1, WC)
    s2_ref[...] = jnp.sum(y * y, axis=0).reshape(1, 1, WC)


def _bn_apply_kernel(y_ref, scale_ref, shift_ref, out_ref):
    out_ref[...] = y_ref[...].astype(jnp.float32) * scale_ref[...] + shift_ref[...]


def _quarter_weights(w_deconv, W, QW, PW):
    C = w_deconv.shape[0]
    Q = W // QW
    halo = (PW - QW) // 2
    wc = jnp.transpose(jnp.flip(w_deconv, axis=(2, 3)), (2, 3, 0, 1))
    wc = wc.astype(jnp.float32)
    band = np.zeros((Q, PW, QW, 3), dtype=np.float32)
    for q in range(Q):
        for wi_l in range(PW):
            wi_g = q * QW - halo + wi_l
            if not (0 <= wi_g < W):
                continue
            for wo_l in range(QW):
                kw = wi_g - (q * QW + wo_l) + 1
                if 0 <= kw < 3:
                    band[q, wi_l, wo_l, kw] = 1.0
    t = jnp.einsum('qiok,hkcd->qhicod', jnp.asarray(band), wc)
    return t.reshape(Q, 3 * PW * C, QW * C).astype(jnp.bfloat16)


@jax.jit
def _forward(x_nchw, w_deconv, gamma, beta):
    N, C, H, W = x_nchw.shape
    WC = W * C
    NB = 32
    while N % NB:
        NB //= 2
    G = N // NB

    QW = 4 if W % 4 == 0 else W
    PW = QW + 4 if QW != W else W
    Q = W // QW

    x2 = jnp.transpose(x_nchw, (0, 2, 3, 1)).reshape(N, H, WC)
    x2 = x2.astype(jnp.bfloat16)
    t = _quarter_weights(w_deconv, W, QW, PW)

    cparams = pltpu.CompilerParams(
        dimension_semantics=("parallel",),
        vmem_limit_bytes=64 * 1024 * 1024,
    )

    conv_kernel = functools.partial(_conv_stats_kernel, H=H, C=C, QW=QW, PW=PW)
    y, s1, s2 = pl.pallas_call(
        conv_kernel,
        grid=(G,),
        in_specs=[
            pl.BlockSpec((NB, H, WC), lambda g: (g, 0, 0)),
            pl.BlockSpec((Q, 3 * PW * C, QW * C), lambda g: (0, 0, 0)),
        ],
        out_specs=(
            pl.BlockSpec((NB, H, WC), lambda g: (g, 0, 0)),
            pl.BlockSpec((1, 1, WC), lambda g: (g, 0, 0)),
            pl.BlockSpec((1, 1, WC), lambda g: (g, 0, 0)),
        ),
        out_shape=(
            jax.ShapeDtypeStruct((N, H, WC), jnp.bfloat16),
            jax.ShapeDtypeStruct((G, 1, WC), jnp.float32),
            jax.ShapeDtypeStruct((G, 1, WC), jnp.float32),
        ),
        compiler_params=cparams,
    )(x2, t)

    m_total = float(N * H * W)
    sum_c = jnp.sum(s1.reshape(G, W, C), axis=(0, 1))
    sq_c = jnp.sum(s2.reshape(G, W, C), axis=(0, 1))
    mean = sum_c / m_total
    var = jnp.maximum(sq_c / m_total - mean * mean, 0.0)
    inv = jax.lax.rsqrt(var + _EPS)
    scale_c = gamma.astype(jnp.float32) * inv
    shift_c = beta.astype(jnp.float32) - mean * scale_c
    scale_wc = jnp.tile(scale_c, W).reshape(1, WC)
    shift_wc = jnp.tile(shift_c, W).reshape(1, WC)

    out2 = pl.pallas_call(
        _bn_apply_kernel,
        grid=(G,),
        in_specs=[
            pl.BlockSpec((NB, H, WC), lambda g: (g, 0, 0)),
            pl.BlockSpec((1, WC), lambda g: (0, 0)),
            pl.BlockSpec((1, WC), lambda g: (0, 0)),
        ],
        out_specs=pl.BlockSpec((NB, H, WC), lambda g: (g, 0, 0)),
        out_shape=jax.ShapeDtypeStruct((N, H, WC), jnp.float32),
        compiler_params=cparams,
    )(y, scale_wc, shift_wc)

    return jnp.transpose(out2.reshape(N, H, W, C), (0, 3, 1, 2))


def kernel(x_nchw, w_deconv, gamma, beta):
    return _forward(x_nchw, w_deconv, gamma, beta)


# speedup vs baseline: 5.9139x; 1.3515x over previous
import functools

import jax
import jax.numpy as jnp
from jax.experimental import pallas as pl
from jax.experimental.pallas import tpu as pltpu

_SLOPE = 0.02
_EPS = 1e-5


def _conv_stats_kernel(x_ref, w_ref, y_ref, s1_ref, s2_ref, *, H, W):
    NB, C, HW = x_ref.shape

    x = x_ref[...].astype(jnp.bfloat16)
    a = jnp.where(x >= 0, x, x * _SLOPE)

    p = jax.lax.broadcasted_iota(jnp.int32, (1, 1, HW), 2)
    w_of_p = p % W

    segs = []
    for kh in range(3):
        for kw in range(3):
            d = (kh - 1) * W + (kw - 1)
            if d > 0:
                s = jnp.concatenate(
                    [a[:, :, d:], jnp.zeros((NB, C, d), a.dtype)], axis=2)
            elif d < 0:
                s = jnp.concatenate(
                    [jnp.zeros((NB, C, -d), a.dtype), a[:, :, :d]], axis=2)
            else:
                s = a
            m = jnp.ones((1, 1, HW), dtype=jnp.bool_)
            if kh == 0:
                m = m & (p >= W)
            elif kh == 2:
                m = m & (p < HW - W)
            if kw == 0:
                m = m & (w_of_p >= 1)
            elif kw == 2:
                m = m & (w_of_p < W - 1)
            segs.append(jnp.where(m, s, jnp.zeros_like(s)))
    a9 = jnp.concatenate(segs, axis=1)

    w9 = w_ref[...]
    s1 = jnp.zeros((C, HW), jnp.float32)
    s2 = jnp.zeros((C, HW), jnp.float32)
    for i in range(NB):
        y = jax.lax.dot_general(
            w9, a9[i], (((0,), (0,)), ((), ())),
            preferred_element_type=jnp.float32)
        y_ref[i] = y.astype(y_ref.dtype)
        s1 = s1 + y
        s2 = s2 + y * y
    s1_ref[...] = s1.reshape(1, C, HW)
    s2_ref[...] = s2.reshape(1, C, HW)


def _bn_apply_kernel(y_ref, scale_ref, shift_ref, out_ref):
    out_ref[...] = (y_ref[...].astype(jnp.float32) * scale_ref[...]
                    + shift_ref[...])


@jax.jit
def _forward(x_nchw, w_deconv, gamma, beta):
    N, C, H, W = x_nchw.shape
    HW = H * W
    NB = 16
    while N % NB:
        NB //= 2
    G = N // NB

    x3 = x_nchw.reshape(N, C, HW)
    wc = jnp.transpose(jnp.flip(w_deconv, axis=(2, 3)), (2, 3, 0, 1))
    w9 = wc.reshape(9 * C, C).astype(jnp.bfloat16)

    cparams = pltpu.CompilerParams(
        dimension_semantics=("parallel",),
        vmem_limit_bytes=64 * 1024 * 1024,
    )

    conv_kernel = functools.partial(_conv_stats_kernel, H=H, W=W)
    y, s1, s2 = pl.pallas_call(
        conv_kernel,
        grid=(G,),
        in_specs=[
            pl.BlockSpec((NB, C, HW), lambda g: (g, 0, 0)),
            pl.BlockSpec((9 * C, C), lambda g: (0, 0)),
        ],
        out_specs=(
            pl.BlockSpec((NB, C, HW), lambda g: (g, 0, 0)),
            pl.BlockSpec((1, C, HW), lambda g: (g, 0, 0)),
            pl.BlockSpec((1, C, HW), lambda g: (g, 0, 0)),
        ),
        out_shape=(
            jax.ShapeDtypeStruct((N, C, HW), jnp.bfloat16),
            jax.ShapeDtypeStruct((G, C, HW), jnp.float32),
            jax.ShapeDtypeStruct((G, C, HW), jnp.float32),
        ),
        compiler_params=cparams,
    )(x3, w9)

    m_total = float(N * H * W)
    sum_c = jnp.sum(s1, axis=(0, 2))
    sq_c = jnp.sum(s2, axis=(0, 2))
    mean = sum_c / m_total
    var = jnp.maximum(sq_c / m_total - mean * mean, 0.0)
    inv = jax.lax.rsqrt(var + _EPS)
    scale_c = gamma.astype(jnp.float32) * inv
    shift_c = beta.astype(jnp.float32) - mean * scale_c
    scale_b = jnp.broadcast_to(scale_c[:, None], (C, HW))
    shift_b = jnp.broadcast_to(shift_c[:, None], (C, HW))

    out3 = pl.pallas_call(
        _bn_apply_kernel,
        grid=(G,),
        in_specs=[
            pl.BlockSpec((NB, C, HW), lambda g: (g, 0, 0)),
            pl.BlockSpec((C, HW), lambda g: (0, 0)),
            pl.BlockSpec((C, HW), lambda g: (0, 0)),
        ],
        out_specs=pl.BlockSpec((NB, C, HW), lambda g: (g, 0, 0)),
        out_shape=jax.ShapeDtypeStruct((N, C, HW), jnp.float32),
        compiler_params=cparams,
    )(y, scale_b, shift_b)

    return out3.reshape(N, C, H, W)


def kernel(x_nchw, w_deconv, gamma, beta):
    return _forward(x_nchw, w_deconv, gamma, beta)
